# baseline (speedup 1.0000x reference)
import functools

import jax
import jax.numpy as jnp
from jax import lax
from jax.experimental import pallas as pl
from jax.experimental.pallas import tpu as pltpu
from jax.experimental.pallas import tpu_sc as plsc

NC = 2
NS = 16
NW = NC * NS
LANES = 16


def _indirect_gather(table_hbm, idx_ref, dst_ref, sem):
    return pltpu.async_copy(table_hbm.at[idx_ref], dst_ref, sem)


def _worker_id():
    return lax.axis_index("s") * NC + lax.axis_index("c")


@functools.lru_cache(maxsize=None)
def _build_relayout(vocab, d):
    n_rows = vocab // 4
    units_full = n_rows // 32
    tail_rows = n_rows - units_full * 32
    base_u, extra_u = divmod(units_full, NW)

    mesh = plsc.VectorSubcoreMesh(
        core_axis_name="c", subcore_axis_name="s",
        num_cores=NC, num_subcores=NS)

    def body(wt3_hbm, out_hbm, slabA, slabB, obufA, obufB, tails,
             isA, isB, osA, osB):
        slab = (slabA, slabB)
        obuf = (obufA, obufB)
        isem = (isA, isB)
        osem = (osA, osB)
        wid = _worker_id()
        u0 = wid * base_u + jnp.minimum(wid, extra_u)
        cnt = base_u + (wid < extra_u).astype(jnp.int32)

        lane = lax.broadcasted_iota(jnp.int32, (LANES,), 0)
        c8a = lax.shift_right_logical(lane, 3)
        csa = lane & 7
        c8b = c8a + 2

        def fire_in(u, b):
            return pltpu.async_copy(
                wt3_hbm.at[:, :, pl.ds(u * 128, 128)], slab[b], isem[b])

        def wait_in(u, b):
            pltpu.make_async_copy(
                wt3_hbm.at[:, :, pl.ds(u * 128, 128)], slab[b], isem[b]).wait()

        def assemble(b):
            def row(j, carry):
                for k in range(8):
                    lv = jnp.zeros((LANES,), jnp.int32) + (4 * j + (k >> 1))
                    c8 = c8a if k % 2 == 0 else c8b
                    vals = plsc.load_gather(slab[b], [c8, csa, lv])
                    obuf[b][j, pl.ds(k * LANES, LANES)] = vals
                return carry

            lax.fori_loop(0, 32, row, 0)

        def out_dst(u):
            return out_hbm.at[pl.ds(u * 32, 32)]

        def body_i(i, carry):
            u = u0 + i
            b = (i % 2).astype(jnp.int32)

            @pl.when(b == 0)
            def _():
                @pl.when(i > 0)
                def _():
                    pltpu.make_async_copy(obufA, out_dst(u - 2), osA).wait()
                @pl.when(i == 0)
                def _():
                    fire_in(u, 0)
                wait_in(u, 0)

                @pl.when(i + 1 < cnt)
                def _():
                    fire_in(u + 1, 1)
                assemble(0)
                pltpu.async_copy(obufA, out_dst(u), osA)

            @pl.when(b == 1)
            def _():
                @pl.when(i > 1)
                def _():
                    pltpu.make_async_copy(obufB, out_dst(u - 2), osB).wait()
                wait_in(u, 1)

                @pl.when(i + 1 < cnt)
                def _():
                    fire_in(u + 1, 0)
                assemble(1)
                pltpu.async_copy(obufB, out_dst(u), osB)
            return carry

        lax.fori_loop(0, cnt, body_i, 0)
        @pl.when(cnt >= 2)
        def _():
            ulast = u0 + cnt - 2
            blast = ((cnt - 2) % 2).astype(jnp.int32)

            @pl.when(blast == 0)
            def _():
                pltpu.make_async_copy(obufA, out_dst(ulast), osA).wait()

            @pl.when(blast == 1)
            def _():
                pltpu.make_async_copy(obufB, out_dst(ulast), osB).wait()

        @pl.when(cnt >= 1)
        def _():
            ulast = u0 + cnt - 1
            blast = ((cnt - 1) % 2).astype(jnp.int32)

            @pl.when(blast == 0)
            def _():
                pltpu.make_async_copy(obufA, out_dst(ulast), osA).wait()

            @pl.when(blast == 1)
            def _():
                pltpu.make_async_copy(obufB, out_dst(ulast), osB).wait()

        if tail_rows:
            @pl.when(wid == NW - 1)
            def _():
                nv = 4 * tail_rows
                for c8 in range(4):
                    for cs in range(8):
                        pltpu.sync_copy(
                            wt3_hbm.at[c8, cs, pl.ds(units_full * 128, nv)],
                            tails.at[c8 * 8 + cs])

                def trow(j, carry):
                    for k in range(8):
                        cvec = lane + 16 * (k % 2)
                        lv = (jnp.zeros((LANES,), jnp.int32)
                              + (4 * j + (k >> 1)))
                        vals = plsc.load_gather(tails, [cvec, lv])
                        obufA[j, pl.ds(k * LANES, LANES)] = vals
                    return carry

                lax.fori_loop(0, tail_rows, trow, 0)
                pltpu.sync_copy(
                    obufA.at[pl.ds(0, tail_rows)],
                    out_hbm.at[pl.ds(units_full * 32, tail_rows)])

    return pl.kernel(
        body,
        out_type=jax.ShapeDtypeStruct((n_rows, 4 * d), jnp.float32),
        mesh=mesh,
        compiler_params=pltpu.CompilerParams(
            use_tc_tiling_on_sc=True, needs_layout_passes=False),
        scratch_types=[
            pltpu.VMEM((4, 8, 128), jnp.float32),
            pltpu.VMEM((4, 8, 128), jnp.float32),
            pltpu.VMEM((32, 4 * d), jnp.float32),
            pltpu.VMEM((32, 4 * d), jnp.float32),
            pltpu.VMEM((32, 64), jnp.float32),
            pltpu.SemaphoreType.DMA,
            pltpu.SemaphoreType.DMA,
            pltpu.SemaphoreType.DMA,
            pltpu.SemaphoreType.DMA,
        ],
    )


@functools.lru_cache(maxsize=None)
def _build(bsz, seq, vocab, d):
    bpw = bsz // NW
    assert bsz % NW == 0 and d == 32 and bpw % LANES == 0
    groups = bpw // LANES

    mesh = plsc.VectorSubcoreMesh(
        core_axis_name="c", subcore_axis_name="s",
        num_cores=NC, num_subcores=NS)

    def body(ids_hbm, table_hbm, extra_hbm, out_hbm,
             idx_v, idx8, rawA, rawB, outA, outB, extra_v,
             gsA, gsB, osA, osB):
        raw = (rawA, rawB)
        outb = (outA, outB)
        gsem = (gsA, gsB)
        osem = (osA, osB)
        wid = _worker_id()
        b0 = wid * bpw

        pltpu.sync_copy(ids_hbm.at[:, pl.ds(b0, bpw)], idx_v)
        pltpu.sync_copy(extra_hbm, extra_v)

        def mkidx(j, carry):
            s = j // groups
            k = j % groups
            v = idx_v[s, pl.ds(k * LANES, LANES)]
            idx8[s, pl.ds(k * LANES, LANES)] = lax.shift_right_logical(v, 2)
            return carry

        lax.fori_loop(0, seq * groups, mkidx, 0)

        def fire(s, b):
            return _indirect_gather(table_hbm, idx8.at[s], raw[b], gsem[b])

        def reserved_hits(v):
            q = lax.shift_right_logical(v, 2)
            return ((q == 0) | (q == 25)).astype(jnp.int32)

        def extract(s, b):
            def grp(t, carry):
                toks = t * LANES + lax.broadcasted_iota(jnp.int32, (LANES,), 0)
                v = idx_v[s, pl.ds(t * LANES, LANES)]
                colbase = lax.shift_left(v & 3, 5)
                for c in range(d):
                    vals = plsc.load_gather(raw[b], [toks, colbase + c])
                    outb[b][c, pl.ds(t * LANES, LANES)] = vals
                hv = reserved_hits(v)

                @pl.when(jnp.max(hv) > 0)
                def _():
                    m = hv != 0
                    e = jnp.where(v < 4, v, v - 96)
                    e = jnp.clip(e, 0, 7)
                    for c in range(d):
                        cv = jnp.zeros((LANES,), jnp.int32) + c
                        fv = plsc.load_gather(extra_v, [e, cv], mask=m)
                        plsc.store_scatter(outb[b], [cv, toks], fv, mask=m)
                return carry

            lax.fori_loop(0, groups, grp, 0)

        def out_dst(s):
            return out_hbm.at[s, :, pl.ds(b0, bpw)]

        def fire_out(s, b):
            return pltpu.async_copy(outb[b], out_dst(s), osem[b])

        def wait_out(s, b):
            pltpu.make_async_copy(outb[b], out_dst(s), osem[b]).wait()

        def chunk_pair(i, carry):
            s0 = i * 2
            s1 = s0 + 1

            @pl.when(i > 0)
            def _():
                wait_out(s1 - 2, 1)
            fire(s1, 1)
            pltpu.make_async_copy(
                table_hbm.at[idx8.at[s0]], raw[0], gsem[0]).wait()
            extract(s0, 0)
            fire_out(s0, 0)

            @pl.when(i < seq // 2 - 1)
            def _():
                wait_out(s0, 0)
                fire(s0 + 2, 0)
            pltpu.make_async_copy(
                table_hbm.at[idx8.at[s1]], raw[1], gsem[1]).wait()
            extract(s1, 1)
            fire_out(s1, 1)
            return carry

        fire(0, 0)
        lax.fori_loop(0, seq // 2, chunk_pair, 0)
        wait_out(seq - 2, 0)
        wait_out(seq - 1, 1)

    return pl.kernel(
        body,
        out_type=jax.ShapeDtypeStruct((seq, d, bsz), jnp.float32),
        mesh=mesh,
        compiler_params=pltpu.CompilerParams(
            use_tc_tiling_on_sc=True, needs_layout_passes=False),
        scratch_types=[
            pltpu.VMEM((seq, bpw), jnp.int32),
            pltpu.VMEM((seq, bpw), jnp.int32),
            pltpu.VMEM((bpw, 4 * d), jnp.float32),
            pltpu.VMEM((bpw, 4 * d), jnp.float32),
            pltpu.VMEM((d, bpw), jnp.float32),
            pltpu.VMEM((d, bpw), jnp.float32),
            pltpu.VMEM((8, d), jnp.float32),
            pltpu.SemaphoreType.DMA,
            pltpu.SemaphoreType.DMA,
            pltpu.SemaphoreType.DMA,
            pltpu.SemaphoreType.DMA,
        ],
    )


def kernel(input_ids, weight, extra_embeddings):
    bsz, seq = input_ids.shape
    vocab, d = weight.shape
    ids_t = input_ids.T
    wt3 = weight.T.reshape(4, d // 4, vocab)
    w4 = _build_relayout(vocab, d)(wt3)
    p = _build(bsz, seq, vocab, d)(ids_t, w4, extra_embeddings)
    return p.transpose(2, 0, 1)


# speedup vs baseline: 2.8432x; 2.8432x over previous
import functools

import jax
import jax.numpy as jnp
from jax import lax
from jax.experimental import pallas as pl
from jax.experimental.pallas import tpu as pltpu
from jax.experimental.pallas import tpu_sc as plsc

NC = 2
NS = 16
NW = NC * NS
LANES = 16


def _indirect_gather(table_hbm, idx_ref, dst_ref, sem):
    return pltpu.async_copy(table_hbm.at[idx_ref], dst_ref, sem)


def _worker_id():
    return lax.axis_index("s") * NC + lax.axis_index("c")


@functools.lru_cache(maxsize=None)
def _build_relayout(vocab, d):
    n_rows = vocab // 4
    U = 4
    rows_u = 32 * U
    lanes_u = 128 * U
    units = vocab // lanes_u
    tail_v = vocab - units * lanes_u
    tail_rows = tail_v // 4
    base_u, extra_u = divmod(units, NW)

    mesh = plsc.VectorSubcoreMesh(
        core_axis_name="c", subcore_axis_name="s",
        num_cores=NC, num_subcores=NS)

    def body(wt_hbm, out_hbm, slabA, slabB, obufA, obufB, tails,
             isA, isB, osA, osB):
        slab = (slabA, slabB)
        obuf = (obufA, obufB)
        isem = (isA, isB)
        osem = (osA, osB)
        wid = _worker_id()
        u0 = wid * base_u + jnp.minimum(wid, extra_u)
        cnt = base_u + (wid < extra_u).astype(jnp.int32)

        lane = lax.broadcasted_iota(jnp.int32, (LANES,), 0)
        lanequad = lax.shift_right_logical(lane, 2)
        lanem36 = (lane & 3) * 36

        def fire_in(u, b):
            for c8 in range(4):
                pltpu.async_copy(
                    wt_hbm.at[pl.ds(c8 * 8, 8), pl.ds(u * lanes_u, lanes_u)],
                    slab[b].at[pl.ds(c8 * 8, 8), pl.ds(0, lanes_u)], isem[b])

        def wait_in(u, b):
            for c8 in range(4):
                pltpu.make_async_copy(
                    wt_hbm.at[pl.ds(c8 * 8, 8), pl.ds(u * lanes_u, lanes_u)],
                    slab[b].at[pl.ds(c8 * 8, 8), pl.ds(0, lanes_u)],
                    isem[b]).wait()

        def assemble(u, b):
            def grpfn(g2, carry):
                for h in range(4):
                    g = g2 * 4 + h
                    lvv = g * LANES + lane
                    jl = g * 4 + lanequad
                    rv = u * rows_u + jl
                    posbase = lanem36 ^ (rv & 127)
                    for c in range(d):
                        cv = jnp.zeros((LANES,), jnp.int32) + c
                        vals = plsc.load_gather(slab[b], [cv, lvv])
                        plsc.store_scatter(obuf[b], [jl, posbase ^ c], vals)
                return carry

            lax.fori_loop(0, lanes_u // LANES // 4, grpfn, 0)

        def out_dst(u):
            return out_hbm.at[pl.ds(u * rows_u, rows_u)]

        def body_i(i, carry):
            u = u0 + i
            b = (i % 2).astype(jnp.int32)

            @pl.when(b == 0)
            def _():
                @pl.when(i > 0)
                def _():
                    pltpu.make_async_copy(obufA, out_dst(u - 2), osA).wait()
                @pl.when(i == 0)
                def _():
                    fire_in(u, 0)
                wait_in(u, 0)

                @pl.when(i + 1 < cnt)
                def _():
                    fire_in(u + 1, 1)
                assemble(u, 0)
                pltpu.async_copy(obufA, out_dst(u), osA)

            @pl.when(b == 1)
            def _():
                @pl.when(i > 1)
                def _():
                    pltpu.make_async_copy(obufB, out_dst(u - 2), osB).wait()
                wait_in(u, 1)

                @pl.when(i + 1 < cnt)
                def _():
                    fire_in(u + 1, 0)
                assemble(u, 1)
                pltpu.async_copy(obufB, out_dst(u), osB)
            return carry

        lax.fori_loop(0, cnt, body_i, 0)

        def drain(off):
            @pl.when(cnt >= off)
            def _():
                ulast = u0 + cnt - off
                blast = ((cnt - off) % 2).astype(jnp.int32)

                @pl.when(blast == 0)
                def _():
                    pltpu.make_async_copy(obufA, out_dst(ulast), osA).wait()

                @pl.when(blast == 1)
                def _():
                    pltpu.make_async_copy(obufB, out_dst(ulast), osB).wait()

        drain(2)
        drain(1)

        if tail_rows:
            @pl.when(wid == NW - 1)
            def _():
                for c in range(d):
                    pltpu.sync_copy(
                        wt_hbm.at[c, pl.ds(units * lanes_u, tail_v)],
                        tails.at[c])

                def trow(j, carry):
                    rot = (units * rows_u + j) & 127
                    rot = jnp.zeros((LANES,), jnp.int32) + rot
                    jv = jnp.zeros((LANES,), jnp.int32) + j
                    for k in range(8):
                        cv = (lane if k % 2 == 0 else lane + 16)
                        lv = (jnp.zeros((LANES,), jnp.int32)
                              + (4 * j + (k >> 1)))
                        vals = plsc.load_gather(tails, [cv, lv])
                        pos = (lane + 16 * (k & 1)) ^ (36 * (k >> 1)) ^ rot
                        plsc.store_scatter(obufA, [jv, pos], vals)
                    return carry

                lax.fori_loop(0, tail_rows, trow, 0)
                pltpu.sync_copy(
                    obufA.at[pl.ds(0, tail_rows)],
                    out_hbm.at[pl.ds(units * rows_u, tail_rows)])

    return pl.kernel(
        body,
        out_type=jax.ShapeDtypeStruct((n_rows, 4 * d), jnp.float32),
        mesh=mesh,
        compiler_params=pltpu.CompilerParams(
            use_tc_tiling_on_sc=True, needs_layout_passes=False),
        scratch_types=[
            pltpu.VMEM((d, lanes_u + 1), jnp.float32),
            pltpu.VMEM((d, lanes_u + 1), jnp.float32),
            pltpu.VMEM((rows_u, 4 * d), jnp.float32),
            pltpu.VMEM((rows_u, 4 * d), jnp.float32),
            pltpu.VMEM((d, 64), jnp.float32),
            pltpu.SemaphoreType.DMA,
            pltpu.SemaphoreType.DMA,
            pltpu.SemaphoreType.DMA,
            pltpu.SemaphoreType.DMA,
        ],
    )


@functools.lru_cache(maxsize=None)
def _build(bsz, seq, vocab, d):
    bpw = bsz // NW
    assert bsz % NW == 0 and d == 32 and bpw % LANES == 0
    groups = bpw // LANES

    mesh = plsc.VectorSubcoreMesh(
        core_axis_name="c", subcore_axis_name="s",
        num_cores=NC, num_subcores=NS)

    def body(ids_hbm, table_hbm, extra_hbm, out_hbm,
             idx_v, idx8, rawA, rawB, outA, outB, extra_v,
             gsA, gsB, osA, osB):
        raw = (rawA, rawB)
        outb = (outA, outB)
        gsem = (gsA, gsB)
        osem = (osA, osB)
        wid = _worker_id()
        b0 = wid * bpw

        pltpu.sync_copy(ids_hbm.at[:, pl.ds(b0, bpw)], idx_v)
        pltpu.sync_copy(extra_hbm, extra_v)

        def mkidx(j, carry):
            s = j // groups
            k = j % groups
            v = idx_v[s, pl.ds(k * LANES, LANES)]
            idx8[s, pl.ds(k * LANES, LANES)] = lax.shift_right_logical(v, 2)
            return carry

        lax.fori_loop(0, seq * groups, mkidx, 0)

        def fire(s, b):
            return _indirect_gather(table_hbm, idx8.at[s], raw[b], gsem[b])

        def reserved_hits(v):
            q = lax.shift_right_logical(v, 2)
            return ((q == 0) | (q == 25)).astype(jnp.int32)

        def extract(s, b):
            def grp(t, carry):
                toks = t * LANES + lax.broadcasted_iota(jnp.int32, (LANES,), 0)
                v = idx_v[s, pl.ds(t * LANES, LANES)]
                cb = ((v & 3) * 36) ^ (lax.shift_right_logical(v, 2) & 127)
                for c in range(d):
                    vals = plsc.load_gather(raw[b], [toks, cb ^ c])
                    outb[b][c, pl.ds(t * LANES, LANES)] = vals
                hv = reserved_hits(v)

                @pl.when(jnp.max(hv) > 0)
                def _():
                    m = hv != 0
                    e = jnp.where(v < 4, v, v - 96)
                    e = jnp.clip(e, 0, 7)
                    for c in range(d):
                        cv = jnp.zeros((LANES,), jnp.int32) + c
                        fv = plsc.load_gather(extra_v, [e, cv], mask=m)
                        plsc.store_scatter(outb[b], [cv, toks], fv, mask=m)
                return carry

            lax.fori_loop(0, groups, grp, 0)

        def out_dst(s):
            return out_hbm.at[s, :, pl.ds(b0, bpw)]

        def fire_out(s, b):
            return pltpu.async_copy(outb[b], out_dst(s), osem[b])

        def wait_out(s, b):
            pltpu.make_async_copy(outb[b], out_dst(s), osem[b]).wait()

        def chunk_pair(i, carry):
            s0 = i * 2
            s1 = s0 + 1

            @pl.when(i > 0)
            def _():
                wait_out(s1 - 2, 1)
            fire(s1, 1)
            pltpu.make_async_copy(
                table_hbm.at[idx8.at[s0]], raw[0], gsem[0]).wait()
            extract(s0, 0)
            fire_out(s0, 0)

            @pl.when(i < seq // 2 - 1)
            def _():
                wait_out(s0, 0)
                fire(s0 + 2, 0)
            pltpu.make_async_copy(
                table_hbm.at[idx8.at[s1]], raw[1], gsem[1]).wait()
            extract(s1, 1)
            fire_out(s1, 1)
            return carry

        fire(0, 0)
        lax.fori_loop(0, seq // 2, chunk_pair, 0)
        wait_out(seq - 2, 0)
        wait_out(seq - 1, 1)

    return pl.kernel(
        body,
        out_type=jax.ShapeDtypeStruct((seq, d, bsz), jnp.float32),
        mesh=mesh,
        compiler_params=pltpu.CompilerParams(
            use_tc_tiling_on_sc=True, needs_layout_passes=False),
        scratch_types=[
            pltpu.VMEM((seq, bpw), jnp.int32),
            pltpu.VMEM((seq, bpw), jnp.int32),
            pltpu.VMEM((bpw, 4 * d), jnp.float32),
            pltpu.VMEM((bpw, 4 * d), jnp.float32),
            pltpu.VMEM((d, bpw), jnp.float32),
            pltpu.VMEM((d, bpw), jnp.float32),
            pltpu.VMEM((8, d), jnp.float32),
            pltpu.SemaphoreType.DMA,
            pltpu.SemaphoreType.DMA,
            pltpu.SemaphoreType.DMA,
            pltpu.SemaphoreType.DMA,
        ],
    )


def kernel(input_ids, weight, extra_embeddings):
    bsz, seq = input_ids.shape
    vocab, d = weight.shape
    ids_t = input_ids.T
    wt = weight.T
    w4 = _build_relayout(vocab, d)(wt)
    p = _build(bsz, seq, vocab, d)(ids_t, w4, extra_embeddings)
    return p.transpose(2, 0, 1)
